# baseline (speedup 1.0000x reference)
import functools

import jax
import jax.numpy as jnp
from jax import lax
from jax.experimental import pallas as pl
from jax.experimental.pallas import tpu as pltpu
from jax.experimental.pallas import tpu_sc as plsc

N_NODES = 10000
N_PAD = 10240
D = 128
E = 320000
EPT = E // 16
CH = 32
NCH = EPT // CH
NB = 8
NI = 10


def _sc_body(nf1_hbm, e1_hbm, e2_hbm, out_hbm, hist_hbm,
             rows, dsti, srci, cloc, gsem, ssem, isem, accum):
    cid = lax.axis_index("c")
    sid = lax.axis_index("s")
    zero16 = jnp.zeros((16,), jnp.float32)
    ones16 = jnp.ones((16,), jnp.float32)

    def _zrow(i, _):
        for j in range(8):
            rows[0, i, pl.ds(16 * j, 16)] = zero16
        return 0
    lax.fori_loop(0, CH, _zrow, 0)

    def _zcloc(i, _):
        cloc[pl.ds(16 * i, 16)] = zero16
        return 0
    lax.fori_loop(0, N_PAD // 16, _zcloc, 0)

    for m in range(640 // CH):
        pltpu.async_copy(rows.at[0], accum.at[pl.ds(640 * sid + CH * m, CH)],
                         gsem.at[m % NB])
    for m in range(640 // CH):
        pltpu.make_async_copy(
            rows.at[0], accum.at[pl.ds(640 * sid + CH * m, CH)],
            gsem.at[m % NB]).wait()

    plsc.subcore_barrier()

    def idx_load(c, ci):
        off = sid * EPT + CH * c

        @pl.when(cid == 0)
        def _():
            pltpu.async_copy(e1_hbm.at[pl.ds(off, CH)], dsti.at[ci],
                             isem.at[ci])
            pltpu.async_copy(e1_hbm.at[pl.ds(E + off, CH)], srci.at[ci],
                             isem.at[ci])

        @pl.when(cid == 1)
        def _():
            pltpu.async_copy(e2_hbm.at[pl.ds(off, CH)], dsti.at[ci],
                             isem.at[ci])
            pltpu.async_copy(e2_hbm.at[pl.ds(E + off, CH)], srci.at[ci],
                             isem.at[ci])

    def idx_wait(ci):
        pltpu.make_async_copy(e1_hbm.at[pl.ds(0, CH)], dsti.at[ci],
                              isem.at[ci]).wait()
        pltpu.make_async_copy(e1_hbm.at[pl.ds(0, CH)], srci.at[ci],
                              isem.at[ci]).wait()

    def gather(b, ci):
        pltpu.async_copy(nf1_hbm.at[srci.at[ci]], rows.at[b], gsem.at[b])

    def gather_wait(b):
        pltpu.make_async_copy(nf1_hbm.at[srci.at[0]], rows.at[b],
                              gsem.at[b]).wait()

    def scat(b, ci):
        pltpu.async_copy(rows.at[b], accum.at[dsti.at[ci]], ssem.at[b],
                         add=True)

    def scat_wait(b):
        pltpu.make_async_copy(rows.at[b], accum.at[dsti.at[0]],
                              ssem.at[b]).wait()

    def counts_upd(ci):
        for k in range(CH // 16):
            v = dsti[ci, pl.ds(16 * k, 16)]
            plsc.addupdate_scatter(cloc, [v], ones16)

    for c0 in range(9):
        idx_load(c0, c0 % NI)
    for c0 in range(7):
        idx_wait(c0)
        gather(c0 % NB, c0)

    def outer(o, _):
        for j in range(40):
            c = 40 * o + j
            b = j % NB
            ci = j % NI

            @pl.when(c < NCH)
            def _():
                gather_wait(b)
                counts_upd(ci)
                scat(b, ci)

            @pl.when(jnp.logical_and(c + 7 < NCH, c >= 1))
            def _():
                scat_wait((b + 7) % NB)

            @pl.when(c + 9 < NCH)
            def _():
                idx_load(c + 9, (j + 9) % NI)

            @pl.when(c + 7 < NCH)
            def _():
                idx_wait((j + 7) % NI)
                gather((j + 7) % NB, (j + 7) % NI)
        return 0

    lax.fori_loop(0, (NCH + 39) // 40, outer, 0)
    for b in range(NB):
        scat_wait(b)

    plsc.subcore_barrier()

    pltpu.sync_copy(accum.at[pl.ds(640 * sid, 640)],
                    out_hbm.at[cid, pl.ds(640 * sid, 640)])
    pltpu.sync_copy(cloc, hist_hbm.at[cid, sid])


_sc_agg = functools.partial(
    pl.kernel,
    out_type=[
        jax.ShapeDtypeStruct((2, N_PAD, D), jnp.float32),
        jax.ShapeDtypeStruct((2, 16, N_PAD), jnp.float32),
    ],
    mesh=plsc.VectorSubcoreMesh(core_axis_name="c", subcore_axis_name="s"),
    compiler_params=pltpu.CompilerParams(needs_layout_passes=False),
    scratch_types=[
        pltpu.VMEM((NB, CH, D), jnp.float32),
        pltpu.VMEM((NI, CH), jnp.int32),
        pltpu.VMEM((NI, CH), jnp.int32),
        pltpu.VMEM((N_PAD,), jnp.float32),
        pltpu.SemaphoreType.DMA((NB,)),
        pltpu.SemaphoreType.DMA((NB,)),
        pltpu.SemaphoreType.DMA((NI,)),
        pltpu.VMEM_SHARED((N_PAD, D), jnp.float32),
    ],
)(_sc_body)


def _tc_body(nf1_ref, nf2_ref, s1_ref, s2_ref, h1_ref, h2_ref, w_ref,
             o_ref):
    r1 = 1.0 / jnp.maximum(jnp.sum(h1_ref[0], axis=0), 1.0)
    r2 = 1.0 / jnp.maximum(jnp.sum(h2_ref[0], axis=0), 1.0)
    acc = jnp.dot(nf1_ref[...], w_ref[0:128, :],
                  preferred_element_type=jnp.float32)
    acc += jnp.dot(nf2_ref[...], w_ref[128:256, :],
                   preferred_element_type=jnp.float32)
    acc += jnp.dot(s1_ref[0] * r1[:, None], w_ref[256:384, :],
                   preferred_element_type=jnp.float32)
    acc += jnp.dot(s2_ref[0] * r2[:, None], w_ref[384:512, :],
                   preferred_element_type=jnp.float32)
    o_ref[...] = jnp.tanh(acc)


_BLK = 1024

_tc_head = pl.pallas_call(
    _tc_body,
    grid=(N_PAD // _BLK,),
    in_specs=[
        pl.BlockSpec((_BLK, D), lambda i: (i, 0)),
        pl.BlockSpec((_BLK, D), lambda i: (i, 0)),
        pl.BlockSpec((1, _BLK, D), lambda i: (0, i, 0)),
        pl.BlockSpec((1, _BLK, D), lambda i: (1, i, 0)),
        pl.BlockSpec((1, 16, _BLK), lambda i: (0, 0, i)),
        pl.BlockSpec((1, 16, _BLK), lambda i: (1, 0, i)),
        pl.BlockSpec((4 * D, D), lambda i: (0, 0)),
    ],
    out_specs=pl.BlockSpec((_BLK, D), lambda i: (i, 0)),
    out_shape=jax.ShapeDtypeStruct((N_NODES, D), jnp.float32),
)


def kernel(node_fea1, node_fea2, edge_index1, edge_index2, weight):
    sums, hists = _sc_agg(node_fea1, edge_index1.reshape(-1),
                          edge_index2.reshape(-1))
    return _tc_head(node_fea1, node_fea2, sums, sums, hists, hists, weight)


# speedup vs baseline: 1.0098x; 1.0098x over previous
import functools

import jax
import jax.numpy as jnp
from jax import lax
from jax.experimental import pallas as pl
from jax.experimental.pallas import tpu as pltpu
from jax.experimental.pallas import tpu_sc as plsc

N_NODES = 10000
N_PAD = 10240
D = 128
E = 320000
EPT = E // 16
CH = 32
NCH = EPT // CH
NB = 8
NI = 10


def _sc_body(nf1_hbm, e1_hbm, e2_hbm, out_hbm, hist_hbm,
             rows, dsti, srci, cloc, gsem, ssem, isem, accum):
    cid = lax.axis_index("c")
    sid = lax.axis_index("s")
    zero16 = jnp.zeros((16,), jnp.float32)
    ones16 = jnp.ones((16,), jnp.float32)

    def _zrow(i, _):
        for j in range(8):
            rows[0, i, pl.ds(16 * j, 16)] = zero16
        return 0
    lax.fori_loop(0, CH, _zrow, 0)

    def _zcloc(i, _):
        cloc[pl.ds(16 * i, 16)] = zero16
        return 0
    lax.fori_loop(0, N_PAD // 16, _zcloc, 0)

    for m in range(640 // CH):
        pltpu.async_copy(rows.at[0], accum.at[pl.ds(640 * sid + CH * m, CH)],
                         gsem.at[m % NB])
    for m in range(640 // CH):
        pltpu.make_async_copy(
            rows.at[0], accum.at[pl.ds(640 * sid + CH * m, CH)],
            gsem.at[m % NB]).wait()

    plsc.subcore_barrier()

    def idx_load(c, ci):
        off = sid * EPT + CH * c

        @pl.when(cid == 0)
        def _():
            pltpu.async_copy(e1_hbm.at[pl.ds(off, CH)], dsti.at[ci],
                             isem.at[ci])
            pltpu.async_copy(e1_hbm.at[pl.ds(E + off, CH)], srci.at[ci],
                             isem.at[ci])

        @pl.when(cid == 1)
        def _():
            pltpu.async_copy(e2_hbm.at[pl.ds(off, CH)], dsti.at[ci],
                             isem.at[ci])
            pltpu.async_copy(e2_hbm.at[pl.ds(E + off, CH)], srci.at[ci],
                             isem.at[ci])

    def idx_wait(ci):
        pltpu.make_async_copy(e1_hbm.at[pl.ds(0, CH)], dsti.at[ci],
                              isem.at[ci]).wait()
        pltpu.make_async_copy(e1_hbm.at[pl.ds(0, CH)], srci.at[ci],
                              isem.at[ci]).wait()

    def gather(b, ci):
        pltpu.async_copy(nf1_hbm.at[srci.at[ci]], rows.at[b], gsem.at[b])

    def gather_wait(b):
        pltpu.make_async_copy(nf1_hbm.at[srci.at[0]], rows.at[b],
                              gsem.at[b]).wait()

    def scat(b, ci):
        pltpu.async_copy(rows.at[b], accum.at[dsti.at[ci]], ssem.at[b],
                         add=True)

    def scat_wait(b):
        pltpu.make_async_copy(rows.at[b], accum.at[dsti.at[0]],
                              ssem.at[b]).wait()

    def counts_upd(ci):
        for k in range(CH // 16):
            v = dsti[ci, pl.ds(16 * k, 16)]
            plsc.addupdate_scatter(cloc, [v], ones16)

    for c0 in range(9):
        idx_load(c0, c0 % NI)
    for c0 in range(7):
        idx_wait(c0)
        gather(c0 % NB, c0)

    def outer(o, _):
        for j in range(40):
            c = 40 * o + j
            b = j % NB
            ci = j % NI

            @pl.when(c < NCH)
            def _():
                gather_wait(b)
                counts_upd(ci)
                scat(b, ci)

            @pl.when(jnp.logical_and(c + 7 < NCH, c >= 1))
            def _():
                scat_wait((b + 7) % NB)

            @pl.when(c + 9 < NCH)
            def _():
                idx_load(c + 9, (j + 9) % NI)

            @pl.when(c + 7 < NCH)
            def _():
                idx_wait((j + 7) % NI)
                gather((j + 7) % NB, (j + 7) % NI)
        return 0

    lax.fori_loop(0, (NCH + 39) // 40, outer, 0)
    for b in range(NB):
        scat_wait(b)

    plsc.subcore_barrier()

    pltpu.sync_copy(accum.at[pl.ds(640 * sid, 640)],
                    out_hbm.at[cid, pl.ds(640 * sid, 640)])
    pltpu.sync_copy(cloc, hist_hbm.at[cid, sid])


_sc_agg = functools.partial(
    pl.kernel,
    out_type=[
        jax.ShapeDtypeStruct((2, N_PAD, D), jnp.float32),
        jax.ShapeDtypeStruct((2, 16, N_PAD), jnp.float32),
    ],
    mesh=plsc.VectorSubcoreMesh(core_axis_name="c", subcore_axis_name="s"),
    compiler_params=pltpu.CompilerParams(needs_layout_passes=False),
    scratch_types=[
        pltpu.VMEM((NB, CH, D), jnp.float32),
        pltpu.VMEM((NI, CH), jnp.int32),
        pltpu.VMEM((NI, CH), jnp.int32),
        pltpu.VMEM((N_PAD,), jnp.float32),
        pltpu.SemaphoreType.DMA((NB,)),
        pltpu.SemaphoreType.DMA((NB,)),
        pltpu.SemaphoreType.DMA((NI,)),
        pltpu.VMEM_SHARED((N_PAD, D), jnp.float32),
    ],
)(_sc_body)


def _tc_body(nf1_ref, nf2_ref, s1_ref, s2_ref, h1_ref, h2_ref, w_ref,
             o_ref):
    r1 = 1.0 / jnp.maximum(jnp.sum(h1_ref[0], axis=0), 1.0)
    r2 = 1.0 / jnp.maximum(jnp.sum(h2_ref[0], axis=0), 1.0)
    acc = jnp.dot(nf1_ref[...], w_ref[0:128, :],
                  preferred_element_type=jnp.float32)
    acc += jnp.dot(nf2_ref[...], w_ref[128:256, :],
                   preferred_element_type=jnp.float32)
    acc += jnp.dot(s1_ref[0] * r1[:, None], w_ref[256:384, :],
                   preferred_element_type=jnp.float32)
    acc += jnp.dot(s2_ref[0] * r2[:, None], w_ref[384:512, :],
                   preferred_element_type=jnp.float32)
    o_ref[...] = jnp.tanh(acc)


_BLK = 2048

_tc_head = pl.pallas_call(
    _tc_body,
    grid=(N_PAD // _BLK,),
    in_specs=[
        pl.BlockSpec((_BLK, D), lambda i: (i, 0)),
        pl.BlockSpec((_BLK, D), lambda i: (i, 0)),
        pl.BlockSpec((1, _BLK, D), lambda i: (0, i, 0)),
        pl.BlockSpec((1, _BLK, D), lambda i: (1, i, 0)),
        pl.BlockSpec((1, 16, _BLK), lambda i: (0, 0, i)),
        pl.BlockSpec((1, 16, _BLK), lambda i: (1, 0, i)),
        pl.BlockSpec((4 * D, D), lambda i: (0, 0)),
    ],
    out_specs=pl.BlockSpec((_BLK, D), lambda i: (i, 0)),
    out_shape=jax.ShapeDtypeStruct((N_NODES, D), jnp.float32),
)


def kernel(node_fea1, node_fea2, edge_index1, edge_index2, weight):
    sums, hists = _sc_agg(node_fea1, edge_index1.reshape(-1),
                          edge_index2.reshape(-1))
    return _tc_head(node_fea1, node_fea2, sums, sums, hists, hists, weight)


# speedup vs baseline: 1.0116x; 1.0018x over previous
import functools

import jax
import jax.numpy as jnp
from jax import lax
from jax.experimental import pallas as pl
from jax.experimental.pallas import tpu as pltpu
from jax.experimental.pallas import tpu_sc as plsc

N_NODES = 10000
N_PAD = 10240
D = 128
E = 320000
EPT = E // 16
CH = 32
NCH = EPT // CH
NB = 8
NI = 10


def _sc_body(nf1_hbm, e1_hbm, e2_hbm, out_hbm, hist_hbm,
             rows, dsti, srci, cloc, gsem, ssem, isem, accum):
    cid = lax.axis_index("c")
    sid = lax.axis_index("s")
    zero16 = jnp.zeros((16,), jnp.float32)
    ones16 = jnp.ones((16,), jnp.float32)

    def _zrow(i, _):
        for j in range(8):
            rows[0, i, pl.ds(16 * j, 16)] = zero16
        return 0
    lax.fori_loop(0, CH, _zrow, 0)

    def _zcloc(i, _):
        cloc[pl.ds(16 * i, 16)] = zero16
        return 0
    lax.fori_loop(0, N_PAD // 16, _zcloc, 0)

    for m in range(640 // CH):
        pltpu.async_copy(rows.at[0], accum.at[pl.ds(640 * sid + CH * m, CH)],
                         gsem.at[m % NB])
    for m in range(640 // CH):
        pltpu.make_async_copy(
            rows.at[0], accum.at[pl.ds(640 * sid + CH * m, CH)],
            gsem.at[m % NB]).wait()

    plsc.subcore_barrier()

    def idx_load(c, ci):
        off = sid * EPT + CH * c

        @pl.when(cid == 0)
        def _():
            pltpu.async_copy(e1_hbm.at[pl.ds(off, CH)], dsti.at[ci],
                             isem.at[ci])
            pltpu.async_copy(e1_hbm.at[pl.ds(E + off, CH)], srci.at[ci],
                             isem.at[ci])

        @pl.when(cid == 1)
        def _():
            pltpu.async_copy(e2_hbm.at[pl.ds(off, CH)], dsti.at[ci],
                             isem.at[ci])
            pltpu.async_copy(e2_hbm.at[pl.ds(E + off, CH)], srci.at[ci],
                             isem.at[ci])

    def idx_wait(ci):
        pltpu.make_async_copy(e1_hbm.at[pl.ds(0, CH)], dsti.at[ci],
                              isem.at[ci]).wait()
        pltpu.make_async_copy(e1_hbm.at[pl.ds(0, CH)], srci.at[ci],
                              isem.at[ci]).wait()

    def gather(b, ci):
        pltpu.async_copy(nf1_hbm.at[srci.at[ci]], rows.at[b], gsem.at[b])

    def gather_wait(b):
        pltpu.make_async_copy(nf1_hbm.at[srci.at[0]], rows.at[b],
                              gsem.at[b]).wait()

    def scat(b, ci):
        pltpu.async_copy(rows.at[b], accum.at[dsti.at[ci]], ssem.at[b],
                         add=True)

    def scat_wait(b):
        pltpu.make_async_copy(rows.at[b], accum.at[dsti.at[0]],
                              ssem.at[b]).wait()

    def counts_upd(ci):
        for k in range(CH // 16):
            v = dsti[ci, pl.ds(16 * k, 16)]
            plsc.addupdate_scatter(cloc, [v], ones16)

    for c0 in range(9):
        idx_load(c0, c0 % NI)
    for c0 in range(7):
        idx_wait(c0)
        gather(c0 % NB, c0)

    def outer(o, _):
        for j in range(40):
            c = 40 * o + j
            b = j % NB
            ci = j % NI

            @pl.when(c < NCH)
            def _():
                gather_wait(b)
                counts_upd(ci)
                scat(b, ci)

            @pl.when(jnp.logical_and(c + 7 < NCH, c >= 1))
            def _():
                scat_wait((b + 7) % NB)

            @pl.when(c + 9 < NCH)
            def _():
                idx_load(c + 9, (j + 9) % NI)

            @pl.when(c + 7 < NCH)
            def _():
                idx_wait((j + 7) % NI)
                gather((j + 7) % NB, (j + 7) % NI)
        return 0

    lax.fori_loop(0, (NCH + 39) // 40, outer, 0)
    pltpu.async_copy(cloc, hist_hbm.at[cid, sid], isem.at[0])
    for b in range(NB):
        scat_wait(b)

    plsc.subcore_barrier()

    pltpu.async_copy(accum.at[pl.ds(640 * sid, 640)],
                     out_hbm.at[cid, pl.ds(640 * sid, 640)], gsem.at[0])
    pltpu.make_async_copy(cloc, hist_hbm.at[cid, sid], isem.at[0]).wait()
    pltpu.make_async_copy(accum.at[pl.ds(640 * sid, 640)],
                          out_hbm.at[cid, pl.ds(640 * sid, 640)],
                          gsem.at[0]).wait()


_sc_agg = functools.partial(
    pl.kernel,
    out_type=[
        jax.ShapeDtypeStruct((2, N_PAD, D), jnp.float32),
        jax.ShapeDtypeStruct((2, 16, N_PAD), jnp.float32),
    ],
    mesh=plsc.VectorSubcoreMesh(core_axis_name="c", subcore_axis_name="s"),
    compiler_params=pltpu.CompilerParams(needs_layout_passes=False),
    scratch_types=[
        pltpu.VMEM((NB, CH, D), jnp.float32),
        pltpu.VMEM((NI, CH), jnp.int32),
        pltpu.VMEM((NI, CH), jnp.int32),
        pltpu.VMEM((N_PAD,), jnp.float32),
        pltpu.SemaphoreType.DMA((NB,)),
        pltpu.SemaphoreType.DMA((NB,)),
        pltpu.SemaphoreType.DMA((NI,)),
        pltpu.VMEM_SHARED((N_PAD, D), jnp.float32),
    ],
)(_sc_body)


def _tc_body(nf1_ref, nf2_ref, s1_ref, s2_ref, h1_ref, h2_ref, w_ref,
             o_ref):
    r1 = 1.0 / jnp.maximum(jnp.sum(h1_ref[0], axis=0), 1.0)
    r2 = 1.0 / jnp.maximum(jnp.sum(h2_ref[0], axis=0), 1.0)
    acc = jnp.dot(nf1_ref[...], w_ref[0:128, :],
                  preferred_element_type=jnp.float32)
    acc += jnp.dot(nf2_ref[...], w_ref[128:256, :],
                   preferred_element_type=jnp.float32)
    acc += jnp.dot(s1_ref[0] * r1[:, None], w_ref[256:384, :],
                   preferred_element_type=jnp.float32)
    acc += jnp.dot(s2_ref[0] * r2[:, None], w_ref[384:512, :],
                   preferred_element_type=jnp.float32)
    o_ref[...] = jnp.tanh(acc)


_BLK = 2048

_tc_head = pl.pallas_call(
    _tc_body,
    grid=(N_PAD // _BLK,),
    in_specs=[
        pl.BlockSpec((_BLK, D), lambda i: (i, 0)),
        pl.BlockSpec((_BLK, D), lambda i: (i, 0)),
        pl.BlockSpec((1, _BLK, D), lambda i: (0, i, 0)),
        pl.BlockSpec((1, _BLK, D), lambda i: (1, i, 0)),
        pl.BlockSpec((1, 16, _BLK), lambda i: (0, 0, i)),
        pl.BlockSpec((1, 16, _BLK), lambda i: (1, 0, i)),
        pl.BlockSpec((4 * D, D), lambda i: (0, 0)),
    ],
    out_specs=pl.BlockSpec((_BLK, D), lambda i: (i, 0)),
    out_shape=jax.ShapeDtypeStruct((N_NODES, D), jnp.float32),
)


def kernel(node_fea1, node_fea2, edge_index1, edge_index2, weight):
    sums, hists = _sc_agg(node_fea1, edge_index1.reshape(-1),
                          edge_index2.reshape(-1))
    return _tc_head(node_fea1, node_fea2, sums, sums, hists, hists, weight)
